# baseline (speedup 1.0000x reference)
import functools

import jax
import jax.numpy as jnp
from jax import lax
from jax.experimental import pallas as pl
from jax.experimental.pallas import tpu as pltpu
from jax.experimental.pallas import tpu_sc as plsc

D = 64
N_VOCAB = 1000000
N_TOK = 819200
NW = 32
NFULL = N_VOCAB // 128
NTAIL = N_VOCAB - NFULL * 128
A_MAIN = NFULL // NW
A_EXTRA = NFULL - A_MAIN * NW
B_BLOCKS = N_TOK // 128 // NW

_mesh = plsc.VectorSubcoreMesh(core_axis_name="c", subcore_axis_name="s")


def _wid():
    return lax.axis_index("s") * 2 + lax.axis_index("c")


def _lanes():
    return jax.lax.broadcasted_iota(jnp.int32, (16,), 0)


def _transpose_col(tile_v, obuf_v):
    lanes = _lanes()

    def kbody(k, carry):
        rot = (lanes + k) & 15
        sbase = rot * D + lanes

        def loads(c16):
            c_idx = lanes + c16 * 16
            return [
                plsc.load_gather(tile_v, [c_idx, rot + j16 * 16])
                for j16 in range(8)
            ]

        def stores(c16, vs):
            sbase_c = sbase + c16 * 16
            for j16 in range(8):
                plsc.store_scatter(
                    obuf_v, [sbase_c + j16 * (16 * D)], vs[j16])

        prev = loads(0)
        for c16 in (1, 2, 3):
            cur = loads(c16)
            stores(c16 - 1, prev)
            prev = cur
        stores(3, prev)
        return carry
    lax.fori_loop(0, 16, kbody, 0)


@functools.partial(
    pl.kernel,
    out_type=jax.ShapeDtypeStruct((N_VOCAB * D,), jnp.float32),
    mesh=_mesh,
    scratch_types=[
        pltpu.VMEM((D, 128), jnp.float32),
        pltpu.VMEM((D, 128), jnp.float32),
        pltpu.VMEM((128 * D,), jnp.float32),
        pltpu.VMEM((128 * D,), jnp.float32),
        pltpu.SemaphoreType.DMA,
        pltpu.SemaphoreType.DMA,
        pltpu.SemaphoreType.DMA,
        pltpu.SemaphoreType.DMA,
    ],
    compiler_params=pltpu.CompilerParams(
        use_tc_tiling_on_sc=True, needs_layout_passes=False),
)
def _detile(pt_hbm, tail_hbm, out_hbm, tile0, tile1, obuf0, obuf1,
            sr0, sr1, sw0, sw1):
    w = _wid()

    def src_at(g):
        return pt_hbm.at[:, pl.ds((w + g * NW) * 128, 128)]

    def dst_at(g):
        return out_hbm.at[pl.ds((w + g * NW) * 128 * D, 128 * D)]

    pltpu.async_copy(src_at(0), tile0, sr0)

    def body(i, carry):
        g0 = 2 * i
        g1 = g0 + 1
        pltpu.make_async_copy(src_at(g0), tile0, sr0).wait()

        @pl.when(g1 < A_MAIN)
        def _():
            pltpu.async_copy(src_at(g1), tile1, sr1)

        @pl.when(i > 0)
        def _():
            pltpu.make_async_copy(obuf0, dst_at(g0 - 2), sw0).wait()
        _transpose_col(tile0, obuf0)
        pltpu.async_copy(obuf0, dst_at(g0), sw0)

        @pl.when(g1 < A_MAIN)
        def _():
            pltpu.make_async_copy(src_at(g1), tile1, sr1).wait()

            @pl.when(g0 + 2 < A_MAIN)
            def _():
                pltpu.async_copy(src_at(g0 + 2), tile0, sr0)

            @pl.when(i > 0)
            def _():
                pltpu.make_async_copy(obuf1, dst_at(g1 - 2), sw1).wait()
            _transpose_col(tile1, obuf1)
            pltpu.async_copy(obuf1, dst_at(g1), sw1)
        return carry

    lax.fori_loop(0, (A_MAIN + 1) // 2, body, 0)
    pltpu.make_async_copy(obuf0, dst_at(A_MAIN - 2), sw0).wait()
    pltpu.make_async_copy(obuf1, dst_at(A_MAIN - 1), sw1).wait()

    @pl.when(w < A_EXTRA)
    def _():
        pltpu.sync_copy(src_at(A_MAIN), tile0)
        _transpose_col(tile0, obuf0)
        pltpu.sync_copy(obuf0, dst_at(A_MAIN))

    @pl.when(w == A_EXTRA)
    def _():
        pltpu.sync_copy(tail_hbm, obuf0.at[pl.ds(0, NTAIL * D)])
        pltpu.sync_copy(
            obuf0.at[pl.ds(0, NTAIL * D)],
            out_hbm.at[pl.ds(NFULL * 128 * D, NTAIL * D)])


@functools.partial(
    pl.kernel,
    out_type=jax.ShapeDtypeStruct((200, 8, 32, 1024), jnp.float32),
    mesh=_mesh,
    scratch_types=[
        pltpu.VMEM((128,), jnp.int32),
        pltpu.VMEM((128,), jnp.int32),
        pltpu.VMEM((128, D), jnp.float32),
        pltpu.VMEM((128, D), jnp.float32),
        pltpu.VMEM((8, 1024), jnp.float32),
        pltpu.VMEM((8, 1024), jnp.float32),
        pltpu.SemaphoreType.DMA,
        pltpu.SemaphoreType.DMA,
        pltpu.SemaphoreType.DMA,
        pltpu.SemaphoreType.DMA,
    ],
    compiler_params=pltpu.CompilerParams(
        use_tc_tiling_on_sc=False, needs_layout_passes=False),
)
def _gather(tok_hbm, table_hbm, out_hbm, idx0, idx1, rows0, rows1,
            t0, t1, sg0, sg1, sw0, sw1):
    bt = _wid()
    lanes = _lanes()

    def tok_at(s):
        return tok_hbm.at[pl.ds(s * 4096 + bt * 128, 128)]

    def out_at(s):
        return out_hbm.at[s, :, bt]

    def transpose_rows(rows_v, t_v):
        def kbody(k, carry):
            rot = (lanes + k) & 15

            def loads(c16):
                c_idx = rot + c16 * 16
                return [
                    plsc.load_gather(rows_v, [lanes + j16 * 16, c_idx])
                    for j16 in range(8)
                ]

            def stores(c16, vs):
                c_idx = rot + c16 * 16
                dt_idx = c_idx >> 3
                ibase = ((c_idx & 7) << 7) + lanes
                for j16 in range(8):
                    plsc.store_scatter(
                        t_v, [dt_idx, ibase + j16 * 16], vs[j16])

            prev = loads(0)
            for c16 in (1, 2, 3):
                cur = loads(c16)
                stores(c16 - 1, prev)
                prev = cur
            stores(3, prev)
            return carry
        lax.fori_loop(0, 16, kbody, 0)

    pltpu.sync_copy(tok_at(0), idx0)
    pltpu.async_copy(table_hbm.at[idx0], rows0, sg0)

    def body(i, carry):
        s0 = 2 * i
        s1 = s0 + 1
        pltpu.sync_copy(tok_at(s1), idx1)
        pltpu.make_async_copy(table_hbm.at[idx0], rows0, sg0).wait()
        pltpu.async_copy(table_hbm.at[idx1], rows1, sg1)

        @pl.when(i > 0)
        def _():
            pltpu.make_async_copy(t0, out_at(s0 - 2), sw0).wait()
        transpose_rows(rows0, t0)
        pltpu.async_copy(t0, out_at(s0), sw0)

        @pl.when(s0 + 2 < B_BLOCKS)
        def _():
            pltpu.sync_copy(tok_at(s0 + 2), idx0)
        pltpu.make_async_copy(table_hbm.at[idx1], rows1, sg1).wait()

        @pl.when(s0 + 2 < B_BLOCKS)
        def _():
            pltpu.async_copy(table_hbm.at[idx0], rows0, sg0)

        @pl.when(i > 0)
        def _():
            pltpu.make_async_copy(t1, out_at(s1 - 2), sw1).wait()
        transpose_rows(rows1, t1)
        pltpu.async_copy(t1, out_at(s1), sw1)
        return carry

    lax.fori_loop(0, B_BLOCKS // 2, body, 0)
    pltpu.make_async_copy(t0, out_at(B_BLOCKS - 2), sw0).wait()
    pltpu.make_async_copy(t1, out_at(B_BLOCKS - 1), sw1).wait()


def kernel(tokens, param):
    tok_t = jnp.transpose(tokens).reshape(-1)
    tail = param[NFULL * 128:, :].reshape(-1)
    flat = _detile(jnp.transpose(param), tail)
    table = flat.reshape(N_VOCAB, D)
    out4 = _gather(tok_t, table)
    out5 = out4.reshape(200, 8, 32, 8, 128)
    return out5.transpose(2, 4, 0, 1, 3).reshape(4096, 200, 64)


# speedup vs baseline: 1.1739x; 1.1739x over previous
import functools

import jax
import jax.numpy as jnp
from jax import lax
from jax.experimental import pallas as pl
from jax.experimental.pallas import tpu as pltpu
from jax.experimental.pallas import tpu_sc as plsc

D = 64
N_VOCAB = 1000000
N_TOK = 819200
NW = 32
NFULL = N_VOCAB // 128
NTAIL = N_VOCAB - NFULL * 128
A_MAIN = NFULL // NW
A_EXTRA = NFULL - A_MAIN * NW
B_BLOCKS = N_TOK // 128 // NW

_mesh = plsc.VectorSubcoreMesh(core_axis_name="c", subcore_axis_name="s")


def _wid():
    return lax.axis_index("s") * 2 + lax.axis_index("c")


def _lanes():
    return jax.lax.broadcasted_iota(jnp.int32, (16,), 0)


def _transpose_col(tile_v, obuf_v):
    lanes = _lanes()

    def kbody(k, carry):
        rot = (lanes + k) & 15
        sbase = rot * D + lanes

        def cbody(c16, carry2):
            c_idx = lanes + c16 * 16
            sbase_c = sbase + c16 * 16
            vs = [
                plsc.load_gather(tile_v, [c_idx, rot + j16 * 16])
                for j16 in range(8)
            ]
            for j16 in range(8):
                plsc.store_scatter(
                    obuf_v, [sbase_c + j16 * (16 * D)], vs[j16])
            return carry2
        return lax.fori_loop(0, 4, cbody, carry, unroll=True)
    lax.fori_loop(0, 16, kbody, 0)


@functools.partial(
    pl.kernel,
    out_type=jax.ShapeDtypeStruct((N_VOCAB * D,), jnp.float32),
    mesh=_mesh,
    scratch_types=[
        pltpu.VMEM((D, 128), jnp.float32),
        pltpu.VMEM((D, 128), jnp.float32),
        pltpu.VMEM((128 * D,), jnp.float32),
        pltpu.VMEM((128 * D,), jnp.float32),
        pltpu.SemaphoreType.DMA,
        pltpu.SemaphoreType.DMA,
        pltpu.SemaphoreType.DMA,
        pltpu.SemaphoreType.DMA,
    ],
    compiler_params=pltpu.CompilerParams(
        use_tc_tiling_on_sc=True, needs_layout_passes=False),
)
def _detile(pt_hbm, tail_hbm, out_hbm, tile0, tile1, obuf0, obuf1,
            sr0, sr1, sw0, sw1):
    w = _wid()

    def src_at(g):
        return pt_hbm.at[:, pl.ds((w + g * NW) * 128, 128)]

    def dst_at(g):
        return out_hbm.at[pl.ds((w + g * NW) * 128 * D, 128 * D)]

    pltpu.async_copy(src_at(0), tile0, sr0)

    def body(i, carry):
        g0 = 2 * i
        g1 = g0 + 1
        pltpu.make_async_copy(src_at(g0), tile0, sr0).wait()

        @pl.when(g1 < A_MAIN)
        def _():
            pltpu.async_copy(src_at(g1), tile1, sr1)

        @pl.when(i > 0)
        def _():
            pltpu.make_async_copy(obuf0, dst_at(g0 - 2), sw0).wait()
        _transpose_col(tile0, obuf0)
        pltpu.async_copy(obuf0, dst_at(g0), sw0)

        @pl.when(g1 < A_MAIN)
        def _():
            pltpu.make_async_copy(src_at(g1), tile1, sr1).wait()

            @pl.when(g0 + 2 < A_MAIN)
            def _():
                pltpu.async_copy(src_at(g0 + 2), tile0, sr0)

            @pl.when(i > 0)
            def _():
                pltpu.make_async_copy(obuf1, dst_at(g1 - 2), sw1).wait()
            _transpose_col(tile1, obuf1)
            pltpu.async_copy(obuf1, dst_at(g1), sw1)
        return carry

    lax.fori_loop(0, (A_MAIN + 1) // 2, body, 0)
    pltpu.make_async_copy(obuf0, dst_at(A_MAIN - 2), sw0).wait()
    pltpu.make_async_copy(obuf1, dst_at(A_MAIN - 1), sw1).wait()

    @pl.when(w < A_EXTRA)
    def _():
        pltpu.sync_copy(src_at(A_MAIN), tile0)
        _transpose_col(tile0, obuf0)
        pltpu.sync_copy(obuf0, dst_at(A_MAIN))

    @pl.when(w == A_EXTRA)
    def _():
        pltpu.sync_copy(tail_hbm, obuf0.at[pl.ds(0, NTAIL * D)])
        pltpu.sync_copy(
            obuf0.at[pl.ds(0, NTAIL * D)],
            out_hbm.at[pl.ds(NFULL * 128 * D, NTAIL * D)])


@functools.partial(
    pl.kernel,
    out_type=jax.ShapeDtypeStruct((200, 8, 32, 1024), jnp.float32),
    mesh=_mesh,
    scratch_types=[
        pltpu.VMEM((200, 128), jnp.int32),
        pltpu.VMEM((128, D), jnp.float32),
        pltpu.VMEM((128, D), jnp.float32),
        pltpu.VMEM((8, 1024), jnp.float32),
        pltpu.VMEM((8, 1024), jnp.float32),
        pltpu.SemaphoreType.DMA,
        pltpu.SemaphoreType.DMA,
        pltpu.SemaphoreType.DMA,
        pltpu.SemaphoreType.DMA,
    ],
    compiler_params=pltpu.CompilerParams(
        use_tc_tiling_on_sc=False, needs_layout_passes=False),
)
def _gather(tok_hbm, table_hbm, out_hbm, idx_all, rows0, rows1,
            t0, t1, sg0, sg1, sw0, sw1):
    bt = _wid()
    lanes = _lanes()

    def out_at(s):
        return out_hbm.at[s, :, bt]

    def transpose_rows(rows_v, t_v):
        def kbody(k, carry):
            rot = (lanes + k) & 15

            def cbody(c16, carry2):
                c_idx = rot + c16 * 16
                dt_idx = c_idx >> 3
                ibase = ((c_idx & 7) << 7) + lanes
                vs = [
                    plsc.load_gather(rows_v, [lanes + j16 * 16, c_idx])
                    for j16 in range(8)
                ]
                for j16 in range(8):
                    plsc.store_scatter(
                        t_v, [dt_idx, ibase + j16 * 16], vs[j16])
                return carry2
            return lax.fori_loop(0, 4, cbody, carry, unroll=True)
        lax.fori_loop(0, 16, kbody, 0)

    pltpu.sync_copy(tok_hbm.at[:, pl.ds(bt * 128, 128)], idx_all)
    pltpu.async_copy(table_hbm.at[idx_all.at[0]], rows0, sg0)

    def body(i, carry):
        s0 = 2 * i
        s1 = s0 + 1
        pltpu.async_copy(table_hbm.at[idx_all.at[s1]], rows1, sg1)
        pltpu.make_async_copy(table_hbm.at[idx_all.at[s0]], rows0, sg0).wait()

        @pl.when(i > 0)
        def _():
            pltpu.make_async_copy(t0, out_at(s0 - 2), sw0).wait()
        transpose_rows(rows0, t0)
        pltpu.async_copy(t0, out_at(s0), sw0)

        @pl.when(s0 + 2 < B_BLOCKS)
        def _():
            pltpu.async_copy(table_hbm.at[idx_all.at[s0 + 2]], rows0, sg0)
        pltpu.make_async_copy(table_hbm.at[idx_all.at[s1]], rows1, sg1).wait()

        @pl.when(i > 0)
        def _():
            pltpu.make_async_copy(t1, out_at(s1 - 2), sw1).wait()
        transpose_rows(rows1, t1)
        pltpu.async_copy(t1, out_at(s1), sw1)
        return carry

    lax.fori_loop(0, B_BLOCKS // 2, body, 0)
    pltpu.make_async_copy(t0, out_at(B_BLOCKS - 2), sw0).wait()
    pltpu.make_async_copy(t1, out_at(B_BLOCKS - 1), sw1).wait()


def kernel(tokens, param):
    tok_t = jnp.transpose(tokens).reshape(200, 4096)
    tail = param[NFULL * 128:, :].reshape(-1)
    flat = _detile(jnp.transpose(param), tail)
    table = flat.reshape(N_VOCAB, D)
    out4 = _gather(tok_t, table)
    out5 = out4.reshape(200, 8, 32, 8, 128)
    return out5.transpose(2, 4, 0, 1, 3).reshape(4096, 200, 64)


# speedup vs baseline: 1.1755x; 1.0014x over previous
import functools

import jax
import jax.numpy as jnp
from jax import lax
from jax.experimental import pallas as pl
from jax.experimental.pallas import tpu as pltpu
from jax.experimental.pallas import tpu_sc as plsc

D = 64
N_VOCAB = 1000000
N_TOK = 819200
NW = 32
NFULL = N_VOCAB // 128
NTAIL = N_VOCAB - NFULL * 128
A_MAIN = NFULL // NW
A_EXTRA = NFULL - A_MAIN * NW
B_BLOCKS = N_TOK // 128 // NW

_mesh = plsc.VectorSubcoreMesh(core_axis_name="c", subcore_axis_name="s")


def _wid():
    return lax.axis_index("s") * 2 + lax.axis_index("c")


def _lanes():
    return jax.lax.broadcasted_iota(jnp.int32, (16,), 0)


def _transpose_col(tile_v, obuf_v):
    lanes = _lanes()

    def kbody(k, carry):
        rot = (lanes + k) & 15
        sbase = rot * D + lanes

        def cbody(c16, carry2):
            c_idx = lanes + c16 * 16
            sbase_c = sbase + c16 * 16
            vs = [
                plsc.load_gather(tile_v, [c_idx, rot + j16 * 16])
                for j16 in range(8)
            ]
            for j16 in range(8):
                plsc.store_scatter(
                    obuf_v, [sbase_c + j16 * (16 * D)], vs[j16])
            return carry2
        return lax.fori_loop(0, 4, cbody, carry, unroll=True)
    lax.fori_loop(0, 16, kbody, 0, unroll=2)


@functools.partial(
    pl.kernel,
    out_type=jax.ShapeDtypeStruct((N_VOCAB * D,), jnp.float32),
    mesh=_mesh,
    scratch_types=[
        pltpu.VMEM((D, 128), jnp.float32),
        pltpu.VMEM((D, 128), jnp.float32),
        pltpu.VMEM((128 * D,), jnp.float32),
        pltpu.VMEM((128 * D,), jnp.float32),
        pltpu.SemaphoreType.DMA,
        pltpu.SemaphoreType.DMA,
        pltpu.SemaphoreType.DMA,
        pltpu.SemaphoreType.DMA,
    ],
    compiler_params=pltpu.CompilerParams(
        use_tc_tiling_on_sc=True, needs_layout_passes=False),
)
def _detile(pt_hbm, tail_hbm, out_hbm, tile0, tile1, obuf0, obuf1,
            sr0, sr1, sw0, sw1):
    w = _wid()

    def src_at(g):
        return pt_hbm.at[:, pl.ds((w + g * NW) * 128, 128)]

    def dst_at(g):
        return out_hbm.at[pl.ds((w + g * NW) * 128 * D, 128 * D)]

    pltpu.async_copy(src_at(0), tile0, sr0)

    def body(i, carry):
        g0 = 2 * i
        g1 = g0 + 1
        pltpu.make_async_copy(src_at(g0), tile0, sr0).wait()

        @pl.when(g1 < A_MAIN)
        def _():
            pltpu.async_copy(src_at(g1), tile1, sr1)

        @pl.when(i > 0)
        def _():
            pltpu.make_async_copy(obuf0, dst_at(g0 - 2), sw0).wait()
        _transpose_col(tile0, obuf0)
        pltpu.async_copy(obuf0, dst_at(g0), sw0)

        @pl.when(g1 < A_MAIN)
        def _():
            pltpu.make_async_copy(src_at(g1), tile1, sr1).wait()

            @pl.when(g0 + 2 < A_MAIN)
            def _():
                pltpu.async_copy(src_at(g0 + 2), tile0, sr0)

            @pl.when(i > 0)
            def _():
                pltpu.make_async_copy(obuf1, dst_at(g1 - 2), sw1).wait()
            _transpose_col(tile1, obuf1)
            pltpu.async_copy(obuf1, dst_at(g1), sw1)
        return carry

    lax.fori_loop(0, (A_MAIN + 1) // 2, body, 0)
    pltpu.make_async_copy(obuf0, dst_at(A_MAIN - 2), sw0).wait()
    pltpu.make_async_copy(obuf1, dst_at(A_MAIN - 1), sw1).wait()

    @pl.when(w < A_EXTRA)
    def _():
        pltpu.sync_copy(src_at(A_MAIN), tile0)
        _transpose_col(tile0, obuf0)
        pltpu.sync_copy(obuf0, dst_at(A_MAIN))

    @pl.when(w == A_EXTRA)
    def _():
        pltpu.sync_copy(tail_hbm, obuf0.at[pl.ds(0, NTAIL * D)])
        pltpu.sync_copy(
            obuf0.at[pl.ds(0, NTAIL * D)],
            out_hbm.at[pl.ds(NFULL * 128 * D, NTAIL * D)])


@functools.partial(
    pl.kernel,
    out_type=jax.ShapeDtypeStruct((200, 8, 32, 1024), jnp.float32),
    mesh=_mesh,
    scratch_types=[
        pltpu.VMEM((200, 128), jnp.int32),
        pltpu.VMEM((128, D), jnp.float32),
        pltpu.VMEM((128, D), jnp.float32),
        pltpu.VMEM((8, 1024), jnp.float32),
        pltpu.VMEM((8, 1024), jnp.float32),
        pltpu.SemaphoreType.DMA,
        pltpu.SemaphoreType.DMA,
        pltpu.SemaphoreType.DMA,
        pltpu.SemaphoreType.DMA,
    ],
    compiler_params=pltpu.CompilerParams(
        use_tc_tiling_on_sc=False, needs_layout_passes=False),
)
def _gather(tok_hbm, table_hbm, out_hbm, idx_all, rows0, rows1,
            t0, t1, sg0, sg1, sw0, sw1):
    bt = _wid()
    lanes = _lanes()

    def out_at(s):
        return out_hbm.at[s, :, bt]

    def transpose_rows(rows_v, t_v):
        def kbody(k, carry):
            rot = (lanes + k) & 15

            def cbody(c16, carry2):
                c_idx = rot + c16 * 16
                dt_idx = c_idx >> 3
                ibase = ((c_idx & 7) << 7) + lanes
                vs = [
                    plsc.load_gather(rows_v, [lanes + j16 * 16, c_idx])
                    for j16 in range(8)
                ]
                for j16 in range(8):
                    plsc.store_scatter(
                        t_v, [dt_idx, ibase + j16 * 16], vs[j16])
                return carry2
            return lax.fori_loop(0, 4, cbody, carry, unroll=True)
        lax.fori_loop(0, 16, kbody, 0, unroll=2)

    pltpu.sync_copy(tok_hbm.at[:, pl.ds(bt * 128, 128)], idx_all)
    pltpu.async_copy(table_hbm.at[idx_all.at[0]], rows0, sg0)

    def body(i, carry):
        s0 = 2 * i
        s1 = s0 + 1
        pltpu.async_copy(table_hbm.at[idx_all.at[s1]], rows1, sg1)
        pltpu.make_async_copy(table_hbm.at[idx_all.at[s0]], rows0, sg0).wait()

        @pl.when(i > 0)
        def _():
            pltpu.make_async_copy(t0, out_at(s0 - 2), sw0).wait()
        transpose_rows(rows0, t0)
        pltpu.async_copy(t0, out_at(s0), sw0)

        @pl.when(s0 + 2 < B_BLOCKS)
        def _():
            pltpu.async_copy(table_hbm.at[idx_all.at[s0 + 2]], rows0, sg0)
        pltpu.make_async_copy(table_hbm.at[idx_all.at[s1]], rows1, sg1).wait()

        @pl.when(i > 0)
        def _():
            pltpu.make_async_copy(t1, out_at(s1 - 2), sw1).wait()
        transpose_rows(rows1, t1)
        pltpu.async_copy(t1, out_at(s1), sw1)
        return carry

    lax.fori_loop(0, B_BLOCKS // 2, body, 0)
    pltpu.make_async_copy(t0, out_at(B_BLOCKS - 2), sw0).wait()
    pltpu.make_async_copy(t1, out_at(B_BLOCKS - 1), sw1).wait()


def kernel(tokens, param):
    tok_t = jnp.transpose(tokens).reshape(200, 4096)
    tail = param[NFULL * 128:, :].reshape(-1)
    flat = _detile(jnp.transpose(param), tail)
    table = flat.reshape(N_VOCAB, D)
    out4 = _gather(tok_t, table)
    out5 = out4.reshape(200, 8, 32, 8, 128)
    return out5.transpose(2, 4, 0, 1, 3).reshape(4096, 200, 64)
